# baseline (speedup 1.0000x reference)
import functools

import jax
import jax.numpy as jnp
from jax import lax
from jax.experimental import pallas as pl
from jax.experimental.pallas import tpu as pltpu
from jax.experimental.pallas import tpu_sc as plsc

LATENT = 32
BATCH = 16384
NROWS = 1000000
SEG = 1 << 17
NSEG = 8
PACK = 4 * LATENT

NC = 2
NS = 16
NW = NC * NS
B_PER_W = BATCH // NW
CHUNK = 128
NCHUNK = B_PER_W // CHUNK

TBLK = 8192
TGRID_I = SEG // TBLK
MLP_BLK = 4096


def _tr_body(*refs):
    out_ref = refs[-1]
    i = pl.program_id(0)
    xcat = jnp.concatenate([r[...] for r in refs[:-1]], axis=0)
    seg = jax.lax.broadcasted_iota(jnp.int32, (2 * PACK, TBLK), 0) // LATENT
    lane = (seg * SEG + i * TBLK
            + jax.lax.broadcasted_iota(jnp.int32, (2 * PACK, TBLK), 1))
    xcat = jnp.where(lane < NROWS, xcat, 0.0)
    u = jax.lax.bitcast_convert_type(xcat, jnp.uint32)
    rb = (u + jnp.uint32(0x8000)) >> 16
    w = rb[:PACK, :] | (rb[PACK:, :] << 16)
    out_ref[...] = jax.lax.bitcast_convert_type(w.T, jnp.float32)


@jax.jit
def _tc_transpose(t):
    last_blk = (NROWS - 1) // TBLK
    def spec(k):
        return pl.BlockSpec(
            (LATENT, TBLK),
            lambda i, _k=k: (0, jnp.minimum(_k * TGRID_I + i, last_blk)))
    return pl.pallas_call(
        _tr_body,
        grid=(TGRID_I,),
        in_specs=[spec(k) for k in range(NSEG)],
        out_specs=pl.BlockSpec((TBLK, PACK), lambda i: (i, 0)),
        out_shape=jax.ShapeDtypeStruct((SEG, PACK), jnp.float32),
    )(*([t] * NSEG))


def _gather_body(tab, idx_hbm, out, idx_v, rows, sem):
    wid = lax.axis_index("s") * NC + lax.axis_index("c")
    base = wid * B_PER_W
    pltpu.sync_copy(idx_hbm.at[wid], idx_v)
    copies = []
    for j in range(NCHUNK):
        copies.append(pltpu.async_copy(
            tab.at[idx_v.at[j]], rows.at[pl.ds(j * CHUNK, CHUNK)], sem))
    for c in copies:
        c.wait()
    pltpu.sync_copy(rows, out.at[pl.ds(base, B_PER_W)])


@jax.jit
def _sc_gather(tab, idx):
    mesh = plsc.VectorSubcoreMesh(core_axis_name="c", subcore_axis_name="s")
    return pl.kernel(
        _gather_body,
        out_type=jax.ShapeDtypeStruct((BATCH, PACK), jnp.float32),
        mesh=mesh,
        scratch_types=[
            pltpu.VMEM((NCHUNK, CHUNK), jnp.int32),
            pltpu.VMEM((B_PER_W, PACK), jnp.float32),
            pltpu.SemaphoreType.DMA,
        ],
    )(tab, idx)


def _mlp_body(xs, xa, inp, w1a4, w1b4, b1, w2, b2, w3, b3, out):
    s = inp[...] >> 17
    seg_lo = jax.lax.broadcasted_iota(jnp.int32, (MLP_BLK, PACK), 1) >> 5
    seg_hi = seg_lo + 4
    us = jax.lax.bitcast_convert_type(xs[...], jnp.uint32)
    ua = jax.lax.bitcast_convert_type(xa[...], jnp.uint32)
    s_lo = jax.lax.bitcast_convert_type(us << 16, jnp.float32)
    s_hi = jax.lax.bitcast_convert_type(us & jnp.uint32(0xFFFF0000), jnp.float32)
    a_lo = jax.lax.bitcast_convert_type(ua << 16, jnp.float32)
    a_hi = jax.lax.bitcast_convert_type(ua & jnp.uint32(0xFFFF0000), jnp.float32)
    xsel = (s_lo * jnp.where(seg_lo == s[:, 0:1], 1.0, 0.0)
            + s_hi * jnp.where(seg_hi == s[:, 0:1], 1.0, 0.0))
    asel = (a_lo * jnp.where(seg_lo == s[:, 1:2], 1.0, 0.0)
            + a_hi * jnp.where(seg_hi == s[:, 1:2], 1.0, 0.0))
    h = jnp.maximum(
        jnp.dot(xsel, w1a4[...], preferred_element_type=jnp.float32)
        + jnp.dot(asel, w1b4[...], preferred_element_type=jnp.float32)
        + b1[...], 0.0)
    h = jnp.maximum(
        jnp.dot(h, w2[...], preferred_element_type=jnp.float32) + b2[...], 0.0)
    out[...] = (jnp.sum(h * w3[...], axis=-1) + b3[...]).reshape(1, MLP_BLK)


@jax.jit
def _tc_mlp(xs, xa, inp, w1a4, w1b4, b1, w2, b2, w3, b3):
    grid = (BATCH // MLP_BLK,)
    full = lambda shape: pl.BlockSpec(shape, lambda i: (0, 0))
    return pl.pallas_call(
        _mlp_body,
        grid=grid,
        in_specs=[
            pl.BlockSpec((MLP_BLK, PACK), lambda i: (i, 0)),
            pl.BlockSpec((MLP_BLK, PACK), lambda i: (i, 0)),
            pl.BlockSpec((MLP_BLK, 2), lambda i: (i, 0)),
            full((PACK, 64)),
            full((PACK, 64)),
            full((1, 64)),
            full((64, LATENT)),
            full((1, LATENT)),
            full((1, LATENT)),
            full((1, 1)),
        ],
        out_specs=pl.BlockSpec((1, MLP_BLK), lambda i: (0, i)),
        out_shape=jax.ShapeDtypeStruct((1, BATCH), jnp.float32),
    )(xs, xa, inp, w1a4, w1b4, b1, w2, b2, w3, b3)


def kernel(inputs, student_table, assessment_table, W1, b1, W2, b2, W3, b3):
    lines = inputs & (SEG - 1)
    idx_s = lines[:, 0].reshape(NW, NCHUNK, CHUNK)
    idx_a = lines[:, 1].reshape(NW, NCHUNK, CHUNK)
    ts = _tc_transpose(student_table.T)
    xs = _sc_gather(ts, idx_s)
    ta = _tc_transpose(assessment_table.T)
    xa = _sc_gather(ta, idx_a)
    return _tc_mlp(
        xs, xa, inputs,
        jnp.tile(W1[:LATENT], (4, 1)), jnp.tile(W1[LATENT:], (4, 1)),
        b1.reshape(1, 64),
        W2, b2.reshape(1, LATENT),
        W3.reshape(1, LATENT), b3.reshape(1, 1),
    ).T


# speedup vs baseline: 1.0047x; 1.0047x over previous
import jax
import jax.numpy as jnp
from jax import lax
from jax.experimental import pallas as pl
from jax.experimental.pallas import tpu as pltpu
from jax.experimental.pallas import tpu_sc as plsc

LATENT = 32
BATCH = 16384
NROWS = 1000000
SEG = 1 << 17
NSEG = 8
PACK = 4 * LATENT

NC = 2
NS = 16
NW = NC * NS
B_PER_W = BATCH // NW
CHUNK = 128
NCHUNK = B_PER_W // CHUNK

TBLK = 8192
TGRID_I = SEG // TBLK
MLP_BLK = 4096


def _tr_body(*refs):
    out_ref = refs[-1]
    i = pl.program_id(0)
    xcat = jnp.concatenate([r[...] for r in refs[:-1]], axis=0)
    seg = jax.lax.broadcasted_iota(jnp.int32, (2 * PACK, TBLK), 0) // LATENT
    lane = (seg * SEG + i * TBLK
            + jax.lax.broadcasted_iota(jnp.int32, (2 * PACK, TBLK), 1))
    xcat = jnp.where(lane < NROWS, xcat, 0.0)
    u = jax.lax.bitcast_convert_type(xcat, jnp.uint32)
    rb = (u + jnp.uint32(0x8000)) >> 16
    w = rb[:PACK, :] | (rb[PACK:, :] << 16)
    out_ref[...] = jax.lax.bitcast_convert_type(w.T, jnp.float32)


@jax.jit
def _tc_transpose(t):
    last_blk = (NROWS - 1) // TBLK
    def spec(k):
        return pl.BlockSpec(
            (LATENT, TBLK),
            lambda i, _k=k: (0, jnp.minimum(_k * TGRID_I + i, last_blk)))
    return pl.pallas_call(
        _tr_body,
        grid=(TGRID_I,),
        in_specs=[spec(k) for k in range(NSEG)],
        out_specs=pl.BlockSpec((TBLK, PACK), lambda i: (i, 0)),
        out_shape=jax.ShapeDtypeStruct((SEG, PACK), jnp.float32),
    )(*([t] * NSEG))


def _gather_body(tab, idx_hbm, out, idx_v, rows, sem):
    wid = lax.axis_index("s") * NC + lax.axis_index("c")
    base = wid * B_PER_W
    pltpu.sync_copy(idx_hbm.at[wid], idx_v)
    copies = []
    for j in range(NCHUNK):
        copies.append(pltpu.async_copy(
            tab.at[idx_v.at[j]], rows.at[pl.ds(j * CHUNK, CHUNK)], sem))
    for c in copies:
        c.wait()
    pltpu.sync_copy(rows, out.at[pl.ds(base, B_PER_W)])


@jax.jit
def _sc_gather(tab, idx):
    mesh = plsc.VectorSubcoreMesh(core_axis_name="c", subcore_axis_name="s")
    return pl.kernel(
        _gather_body,
        out_type=jax.ShapeDtypeStruct((BATCH, PACK), jnp.float32),
        mesh=mesh,
        scratch_types=[
            pltpu.VMEM((NCHUNK, CHUNK), jnp.int32),
            pltpu.VMEM((B_PER_W, PACK), jnp.float32),
            pltpu.SemaphoreType.DMA,
        ],
    )(tab, idx)


def _mlp_body(xs, xa, inp, w1a4, w1b4, b1, w2, b2, w3, b3, out):
    s = inp[...] >> 17
    seg_lo = jax.lax.broadcasted_iota(jnp.int32, (MLP_BLK, PACK), 1) >> 5
    seg_hi = seg_lo + 4
    us = jax.lax.bitcast_convert_type(xs[...], jnp.uint32)
    ua = jax.lax.bitcast_convert_type(xa[...], jnp.uint32)
    s_lo = jax.lax.bitcast_convert_type(us << 16, jnp.float32)
    s_hi = jax.lax.bitcast_convert_type(us & jnp.uint32(0xFFFF0000), jnp.float32)
    a_lo = jax.lax.bitcast_convert_type(ua << 16, jnp.float32)
    a_hi = jax.lax.bitcast_convert_type(ua & jnp.uint32(0xFFFF0000), jnp.float32)
    xsel = (s_lo * jnp.where(seg_lo == s[:, 0:1], 1.0, 0.0)
            + s_hi * jnp.where(seg_hi == s[:, 0:1], 1.0, 0.0))
    asel = (a_lo * jnp.where(seg_lo == s[:, 1:2], 1.0, 0.0)
            + a_hi * jnp.where(seg_hi == s[:, 1:2], 1.0, 0.0))
    h = jnp.maximum(
        jnp.dot(xsel, w1a4[...], preferred_element_type=jnp.float32)
        + jnp.dot(asel, w1b4[...], preferred_element_type=jnp.float32)
        + b1[...], 0.0)
    h = jnp.maximum(
        jnp.dot(h, w2[...], preferred_element_type=jnp.float32) + b2[...], 0.0)
    out[...] = (jnp.sum(h * w3[...], axis=-1) + b3[...]).reshape(1, MLP_BLK)


@jax.jit
def _tc_mlp(xs, xa, inp, w1a4, w1b4, b1, w2, b2, w3, b3):
    grid = (BATCH // MLP_BLK,)
    full = lambda shape: pl.BlockSpec(shape, lambda i: (0, 0))
    return pl.pallas_call(
        _mlp_body,
        grid=grid,
        in_specs=[
            pl.BlockSpec((MLP_BLK, PACK), lambda i: (i, 0)),
            pl.BlockSpec((MLP_BLK, PACK), lambda i: (i, 0)),
            pl.BlockSpec((MLP_BLK, 2), lambda i: (i, 0)),
            full((PACK, 64)),
            full((PACK, 64)),
            full((1, 64)),
            full((64, LATENT)),
            full((1, LATENT)),
            full((1, LATENT)),
            full((1, 1)),
        ],
        out_specs=pl.BlockSpec((1, MLP_BLK), lambda i: (0, i)),
        out_shape=jax.ShapeDtypeStruct((1, BATCH), jnp.float32),
    )(xs, xa, inp, w1a4, w1b4, b1, w2, b2, w3, b3)


def kernel(inputs, student_table, assessment_table, W1, b1, W2, b2, W3, b3):
    lines = inputs & (SEG - 1)
    idx_s = lines[:, 0].reshape(NW, NCHUNK, CHUNK)
    idx_a = lines[:, 1].reshape(NW, NCHUNK, CHUNK)
    ts = _tc_transpose(student_table.T)
    xs = _sc_gather(ts, idx_s)
    ta = _tc_transpose(assessment_table.T)
    xa = _sc_gather(ta, idx_a)
    return _tc_mlp(
        xs, xa, inputs,
        jnp.tile(W1[:LATENT], (4, 1)), jnp.tile(W1[LATENT:], (4, 1)),
        b1.reshape(1, 64),
        W2, b2.reshape(1, LATENT),
        W3.reshape(1, LATENT), b3.reshape(1, 1),
    ).T
